# Initial kernel scaffold; baseline (speedup 1.0000x reference)
import jax
import jax.numpy as jnp
from jax.experimental import pallas as pl


def kernel(x_obs, pos_obs, pos_query, W_enc1, b_enc1, W_enc2, b_enc2, W_self_0, W_nbr_0, b_gnn_0, W_self_1, W_nbr_1, b_gnn_1, W_e1, b_e1, W_e2, b_e2, W_v, b_v, W_o, b_o, W_d1, b_d1, W_d2, b_d2):
    raise NotImplementedError("write your pallas kernel here")


# speedup vs baseline: 2.2787x; 2.2787x over previous
import functools

import jax
import jax.numpy as jnp
from jax import lax
from jax.experimental import pallas as pl
from jax.experimental.pallas import tpu as pltpu
from jax.experimental.pallas import tpu_sc as plsc

N_OBS = 10000
N_QUERY = 10000
PROJ = 128
HEADS = 8
LATENT = 16
K = 16
OUT_DIM = 3
PAD = 10240
E = PAD * K

_info = plsc.get_sparse_core_info()
_NC, _NS = _info.num_cores, _info.num_subcores
_NW = _NC * _NS


def _make_sc_gather(D, chunk):
    per_w = E // _NW
    nch = per_w // chunk
    assert per_w % chunk == 0
    mesh = plsc.VectorSubcoreMesh(core_axis_name="c", subcore_axis_name="s")

    @functools.partial(
        pl.kernel, mesh=mesh,
        out_type=jax.ShapeDtypeStruct((E, D), jnp.float32),
        scratch_types=[
            pltpu.VMEM((chunk,), jnp.int32),
            pltpu.VMEM((chunk, D), jnp.float32),
            pltpu.SemaphoreType.DMA,
        ],
    )
    def k(table_hbm, idx_hbm, out_hbm, idx_v, rows_v, sem):
        wid = lax.axis_index("s") * _NC + lax.axis_index("c")
        for c in range(nch):
            base = wid * per_w + c * chunk
            pltpu.sync_copy(idx_hbm.at[pl.ds(base, chunk)], idx_v)
            pltpu.async_copy(table_hbm.at[idx_v], rows_v, sem).wait()
            pltpu.sync_copy(rows_v, out_hbm.at[pl.ds(base, chunk)])

    return k


_gather128 = _make_sc_gather(128, 512)


def _mm_body(relu, x_ref, w_ref, b_ref, o_ref):
    acc = jnp.dot(x_ref[:], w_ref[:], preferred_element_type=jnp.float32)
    acc = acc + b_ref[:]
    o_ref[:] = jnp.maximum(acc, 0.0) if relu else acc


def _mm(x, w, b, relu=False):
    n = x.shape[0]
    return pl.pallas_call(
        functools.partial(_mm_body, relu),
        grid=(n // 1024,),
        in_specs=[
            pl.BlockSpec((1024, 128), lambda i: (i, 0)),
            pl.BlockSpec((128, 128), lambda i: (0, 0)),
            pl.BlockSpec((1, 128), lambda i: (0, 0)),
        ],
        out_specs=pl.BlockSpec((1024, 128), lambda i: (i, 0)),
        out_shape=jax.ShapeDtypeStruct((n, 128), jnp.float32),
    )(x, w, b.reshape(1, 128))


def _combine_body(h_ref, g_ref, w_ref, b_ref, o_ref):
    msg = jnp.mean(g_ref[:], axis=1)
    acc = jnp.dot(h_ref[:], w_ref[:], preferred_element_type=jnp.float32)
    o_ref[:] = jnp.maximum(acc + msg + b_ref[:], 0.0)


def _combine(h, g, w, b):
    return pl.pallas_call(
        _combine_body,
        grid=(PAD // 1024,),
        in_specs=[
            pl.BlockSpec((1024, 128), lambda i: (i, 0)),
            pl.BlockSpec((1024, K, 128), lambda i: (i, 0, 0)),
            pl.BlockSpec((128, 128), lambda i: (0, 0)),
            pl.BlockSpec((1, 128), lambda i: (0, 0)),
        ],
        out_specs=pl.BlockSpec((1024, 128), lambda i: (i, 0)),
        out_shape=jax.ShapeDtypeStruct((PAD, 128), jnp.float32),
    )(h, g, w, b.reshape(1, 128))


def _knn_body(q_ref, obs_ref, idx_ref, d_ref):
    q = q_ref[:]
    d2 = jnp.zeros((128, PAD), jnp.float32)
    for j in range(3):
        diff = q[:, j:j + 1] - obs_ref[j:j + 1, :]
        d2 = d2 + diff * diff
    d_ref[:] = d2
    col = lax.broadcasted_iota(jnp.int32, (128, PAD), 1)
    big_i = jnp.int32(2 ** 30)
    inf = jnp.float32(3e38)

    lane16 = lax.broadcasted_iota(jnp.int32, (128, K), 1)

    def step(t, im):
        d = d_ref[:]
        m = jnp.min(d, axis=1, keepdims=True)
        cand = jnp.where(d == m, col, big_i)
        amin = jnp.min(cand, axis=1, keepdims=True)
        d_ref[:] = jnp.where(col == amin, inf, d)
        return jnp.where(lane16 == t, amin, im)

    idx_ref[:] = lax.fori_loop(0, K, step, jnp.zeros((128, K), jnp.int32))


def _knn(qpos8, obs_t):
    return pl.pallas_call(
        _knn_body,
        grid=(PAD // 128,),
        in_specs=[
            pl.BlockSpec((128, 8), lambda i: (i, 0)),
            pl.BlockSpec((8, PAD), lambda i: (0, 0)),
        ],
        out_specs=pl.BlockSpec((128, K), lambda i: (i, 0)),
        out_shape=jax.ShapeDtypeStruct((PAD, K), jnp.int32),
        scratch_shapes=[pltpu.VMEM((128, PAD), jnp.float32)],
    )(qpos8, obs_t)


def _attn_body(pq_ref, pn_ref, vg_ref, we1_ref, be1_ref, we2_ref, be2_ref,
               wo_ref, bo_ref, wd1_ref, bd1_ref, wd2_ref, bd2_ref, o_ref):
    B = 256
    pq = pq_ref[:, :3]
    lks = []
    m = None
    for k in range(K):
        pn = pn_ref[:, k, :3]
        rel = pn - pq
        z = (jnp.dot(pq, we1_ref[0:3, :], preferred_element_type=jnp.float32)
             + jnp.dot(pn, we1_ref[3:6, :], preferred_element_type=jnp.float32)
             + jnp.dot(rel, we1_ref[6:9, :], preferred_element_type=jnp.float32)
             + be1_ref[:])
        hk = jnp.maximum(z, 0.0)
        lk = jnp.dot(hk, we2_ref[:], preferred_element_type=jnp.float32) + be2_ref[:]
        lks.append(lk)
        m = lk if m is None else jnp.maximum(m, lk)
    s = jnp.zeros((B, HEADS), jnp.float32)
    agg = jnp.zeros((B, HEADS, LATENT), jnp.float32)
    for k in range(K):
        e = jnp.exp(lks[k] - m)
        s = s + e
        agg = agg + e[:, :, None] * vg_ref[:, k, :, :]
    agg = agg / s[:, :, None]
    hq = jnp.broadcast_to(bo_ref[:], (B, 128))
    for h in range(HEADS):
        hq = hq + jnp.dot(agg[:, h, :], wo_ref[h * 16:(h + 1) * 16, :],
                          preferred_element_type=jnp.float32)
    dd = jnp.maximum(jnp.dot(hq, wd1_ref[:], preferred_element_type=jnp.float32)
                     + bd1_ref[:], 0.0)
    oo = jnp.dot(dd, wd2_ref[:], preferred_element_type=jnp.float32) + bd2_ref[:]
    sp = jnp.maximum(oo, 0.0) + jnp.log(1.0 + jnp.exp(-jnp.abs(oo)))
    ci = lax.broadcasted_iota(jnp.int32, (B, 2 * OUT_DIM), 1)
    o_ref[:] = jnp.where(ci < OUT_DIM, oo, sp)


def _attn(pq8, pn, vg, we1, be1, we2, be2, wo, bo, wd1, bd1, wd2, bd2):
    full = lambda shape: pl.BlockSpec(shape, lambda i: (0,) * len(shape))
    return pl.pallas_call(
        _attn_body,
        grid=(PAD // 256,),
        in_specs=[
            pl.BlockSpec((256, 8), lambda i: (i, 0)),
            pl.BlockSpec((256, K, 128), lambda i: (i, 0, 0)),
            pl.BlockSpec((256, K, HEADS, LATENT), lambda i: (i, 0, 0, 0)),
            full((9, 128)), full((1, 128)),
            full((128, HEADS)), full((1, HEADS)),
            full((128, 128)), full((1, 128)),
            full((128, 128)), full((1, 128)),
            full((128, 2 * OUT_DIM)), full((1, 2 * OUT_DIM)),
        ],
        out_specs=pl.BlockSpec((256, 2 * OUT_DIM), lambda i: (i, 0)),
        out_shape=jax.ShapeDtypeStruct((PAD, 2 * OUT_DIM), jnp.float32),
    )(pq8, pn, vg, we1, be1.reshape(1, 128), we2, be2.reshape(1, HEADS),
      wo, bo.reshape(1, 128), wd1, bd1.reshape(1, 128),
      wd2, bd2.reshape(1, 2 * OUT_DIM))


def kernel(x_obs, pos_obs, pos_query, W_enc1, b_enc1, W_enc2, b_enc2,
           W_self_0, W_nbr_0, b_gnn_0, W_self_1, W_nbr_1, b_gnn_1,
           W_e1, b_e1, W_e2, b_e2, W_v, b_v, W_o, b_o,
           W_d1, b_d1, W_d2, b_d2):
    f32 = jnp.float32
    zeros128 = jnp.zeros((128,), f32)

    x_p = jnp.pad(x_obs, ((0, PAD - N_OBS), (0, 0)))
    obs_t = jnp.pad(pos_obs.T, ((0, 5), (0, PAD - N_OBS)),
                    constant_values=1e4)
    po8 = jnp.pad(pos_obs, ((0, PAD - N_OBS), (0, 5)))
    pq8 = jnp.pad(pos_query, ((0, PAD - N_QUERY), (0, 5)))
    pos128 = jnp.pad(pos_obs, ((0, PAD - N_OBS), (0, 125)))

    h0 = _mm(_mm(x_p, W_enc1, b_enc1, relu=True), W_enc2, b_enc2)

    nbr = _knn(po8, obs_t)
    q_idx = _knn(pq8, obs_t)
    nbr_flat = nbr.reshape(E)
    q_flat = q_idx.reshape(E)

    hn0 = _mm(h0, W_nbr_0, zeros128)
    g0 = _gather128(hn0, nbr_flat).reshape(PAD, K, 128)
    h1 = _combine(h0, g0, W_self_0, b_gnn_0)
    hn1 = _mm(h1, W_nbr_1, zeros128)
    g1 = _gather128(hn1, nbr_flat).reshape(PAD, K, 128)
    h2 = _combine(h1, g1, W_self_1, b_gnn_1)

    v = _mm(h2, W_v, b_v)
    vg = _gather128(v, q_flat).reshape(PAD, K, HEADS, LATENT)
    pn = _gather128(pos128, q_flat).reshape(PAD, K, 128)

    out = _attn(pq8, pn, vg, W_e1, b_e1, W_e2, b_e2,
                W_o, b_o, W_d1, b_d1, W_d2, b_d2)
    return out[:N_QUERY]
